# baseline (speedup 1.0000x reference)
import functools

import jax
import jax.numpy as jnp
from jax import lax
from jax.experimental import pallas as pl
from jax.experimental.pallas import tpu as pltpu
from jax.experimental.pallas import tpu_sc as plsc

D_MODEL = 64
SCALE = 8.0
NC = 2
NS = 16
NW = NC * NS
C = 128


@functools.lru_cache(maxsize=None)
def _make_emb(B):
    bpw = B // NW
    nch = bpw // C
    mesh = plsc.VectorSubcoreMesh(core_axis_name="c", subcore_axis_name="s")

    @functools.partial(
        pl.kernel,
        out_type=jax.ShapeDtypeStruct((B, D_MODEL), jnp.float32),
        mesh=mesh,
        scratch_types=[
            pltpu.VMEM((nch, C), jnp.int32),
            pltpu.VMEM((C, D_MODEL), jnp.float32),
            pltpu.SemaphoreType.DMA,
        ],
        compiler_params=pltpu.CompilerParams(use_tc_tiling_on_sc=False),
    )
    def emb(x_hbm, table_hbm, out_hbm, idx_v, rows, gsem):
        wid = lax.axis_index("s") * NC + lax.axis_index("c")
        base = wid * bpw
        pltpu.sync_copy(x_hbm.at[wid], idx_v)

        def chunk(g, carry):
            pltpu.async_copy(table_hbm.at[idx_v.at[g]], rows, gsem).wait()

            def scale_row(r, carry2):
                for jj in range(D_MODEL // 16):
                    sl = pl.ds(16 * jj, 16)
                    rows[r, sl] = rows[r, sl] * SCALE
                return carry2

            lax.fori_loop(0, C, scale_row, 0)
            pltpu.sync_copy(rows, out_hbm.at[pl.ds(base + g * C, C)])
            return carry

        lax.fori_loop(0, nch, chunk, 0)

    return emb


def kernel(x, table):
    n, s = x.shape
    B = n * s
    idx = x.reshape(NW, B // NW // C, C).astype(jnp.int32)
    out = _make_emb(B)(idx, table)
    return out.reshape(n, s, D_MODEL)


# speedup vs baseline: 1.2078x; 1.2078x over previous
import functools

import jax
import jax.numpy as jnp
from jax import lax
from jax.experimental import pallas as pl
from jax.experimental.pallas import tpu as pltpu
from jax.experimental.pallas import tpu_sc as plsc

D_MODEL = 64
SCALE = 8.0
NC = 2
NS = 16
NW = NC * NS
C = 128
NBUF = 4


@functools.lru_cache(maxsize=None)
def _make_emb(B):
    bpw = B // NW
    nch = bpw // C
    nchj = nch // NBUF
    mesh = plsc.VectorSubcoreMesh(core_axis_name="c", subcore_axis_name="s")

    @functools.partial(
        pl.kernel,
        out_type=jax.ShapeDtypeStruct((B, D_MODEL), jnp.float32),
        mesh=mesh,
        scratch_types=(
            [pltpu.VMEM((nch, C), jnp.int32)]
            + [pltpu.VMEM((C, D_MODEL), jnp.float32)] * (2 * NBUF)
            + [pltpu.SemaphoreType.DMA] * (2 * NBUF)
        ),
        compiler_params=pltpu.CompilerParams(use_tc_tiling_on_sc=False),
    )
    def emb(x_hbm, table_hbm, out_hbm, idx_v, *bufs):
        gbuf = bufs[0:NBUF]
        obuf = bufs[NBUF:2 * NBUF]
        gsem = bufs[2 * NBUF:3 * NBUF]
        osem = bufs[3 * NBUF:4 * NBUF]
        wid = lax.axis_index("s") * NC + lax.axis_index("c")
        base = wid * bpw
        pltpu.sync_copy(x_hbm.at[wid], idx_v)

        for b in range(NBUF):
            pltpu.async_copy(table_hbm.at[idx_v.at[b]], gbuf[b], gsem[b])

        def body(j, carry):
            for b in range(NBUF):
                g = j * NBUF + b
                pltpu.make_async_copy(
                    table_hbm.at[idx_v.at[0]], gbuf[b], gsem[b]).wait()

                @pl.when(j > 0)
                def _():
                    pltpu.make_async_copy(
                        obuf[b], out_hbm.at[pl.ds(0, C)], osem[b]).wait()

                src, dst = gbuf[b], obuf[b]

                @plsc.parallel_loop(0, C, 1, unroll=8)
                def _scale(r):
                    for jj in range(D_MODEL // 16):
                        sl = pl.ds(16 * jj, 16)
                        dst[r, sl] = src[r, sl] * SCALE

                @pl.when(j < nchj - 1)
                def _():
                    pltpu.async_copy(
                        table_hbm.at[idx_v.at[g + NBUF]], gbuf[b], gsem[b])

                pltpu.async_copy(
                    obuf[b], out_hbm.at[pl.ds(base + g * C, C)], osem[b])
            return carry

        lax.fori_loop(0, nchj, body, 0)
        for b in range(NBUF):
            pltpu.make_async_copy(
                obuf[b], out_hbm.at[pl.ds(0, C)], osem[b]).wait()

    return emb


def kernel(x, table):
    n, s = x.shape
    B = n * s
    idx = x.reshape(NW, B // NW // C, C).astype(jnp.int32)
    out = _make_emb(B)(idx, table)
    return out.reshape(n, s, D_MODEL)
